# baseline (speedup 1.0000x reference)
import functools

import jax
import jax.numpy as jnp
from jax import lax
from jax.experimental import pallas as pl
from jax.experimental.pallas import tpu as pltpu
from jax.experimental.pallas import tpu_sc as plsc

N = 10000
E = 320000
D = 128
DP = 144
NC = 2
NS = 16
NW = NC * NS
EPT = E // NW
CH = 64
NCH = 158
EPTP = NCH * CH
NPT = N // NS
LANES = 16


def _node_prep(h, asv, adv, hp_ref, ad_ref, gm_ref):
    a_s = jnp.dot(h, asv, preferred_element_type=jnp.float32)
    a_d = jnp.dot(h, adv, preferred_element_type=jnp.float32)
    cols = lax.broadcasted_iota(jnp.int32, (N, DP - D), 1)
    tail = jnp.where(cols == 0, 1.0,
                     jnp.where(cols == 1, a_s[:, None], 0.0))
    hp_ref[:, :D] = h
    hp_ref[:, D:] = tail.astype(jnp.float32)
    ad_ref[...] = a_d
    gm_ref[...] = jnp.full((D,), jnp.max(a_s), dtype=jnp.float32)


def _prep1_body(x_ref, w_ref, asv_ref, adv_ref, hp_ref, ad_ref, gm_ref):
    h = jnp.dot(x_ref[...], w_ref[...], preferred_element_type=jnp.float32)
    _node_prep(h, asv_ref[...], adv_ref[...], hp_ref, ad_ref, gm_ref)


def _combine(s):
    feat = s[:, :D]
    cols = lax.broadcasted_iota(jnp.int32, (N, DP - D), 1)
    den = jnp.sum(jnp.where(cols == 0, s[:, D:], 0.0), axis=1)
    return feat, den


def _prep2_body(s_ref, b_ref, w_ref, asv_ref, adv_ref,
                hp_ref, ad_ref, gm_ref):
    feat, den = _combine(s_ref[0] + s_ref[1])
    y = feat / (den[:, None] + 1e-16) + b_ref[...][None, :]
    y = jnp.maximum(y, 0.0)
    h = jnp.dot(y, w_ref[...], preferred_element_type=jnp.float32)
    _node_prep(h, asv_ref[...], adv_ref[...], hp_ref, ad_ref, gm_ref)


def _final_body(s_ref, b_ref, out_ref):
    feat, den = _combine(s_ref[0] + s_ref[1])
    out_ref[...] = feat / (den[:, None] + 1e-16) + b_ref[...][None, :]


_node_out = [
    jax.ShapeDtypeStruct((N, DP), jnp.float32),
    jax.ShapeDtypeStruct((N,), jnp.float32),
    jax.ShapeDtypeStruct((D,), jnp.float32),
]

_prep1 = pl.pallas_call(_prep1_body, out_shape=_node_out)
_prep2 = pl.pallas_call(_prep2_body, out_shape=_node_out)
_final = pl.pallas_call(
    _final_body, out_shape=jax.ShapeDtypeStruct((N, D), jnp.float32))


def _edge_body(hp, pk, adst, gmax, out,
               s_acc, ad_l, gm_l, pk_l,
               src0, src1, dst0, dst1, w_l, rows0, rows1,
               gsem, ssem):
    c_id = lax.axis_index("c")
    s_id = lax.axis_index("s")
    wid = s_id * NC + c_id
    base = s_id * NPT

    pltpu.sync_copy(adst, ad_l)
    pltpu.sync_copy(gmax.at[pl.ds(0, LANES)], gm_l)
    pltpu.sync_copy(pk.at[wid], pk_l)

    srcb = (src0, src1)
    dstb = (dst0, dst1)
    rowsb = (rows0, rows1)

    @plsc.parallel_loop(0, CH, 1, unroll=4)
    def zinit(i):
        for k in range(DP // LANES):
            rows0[i, pl.ds(k * LANES, LANES)] = jnp.zeros((LANES,),
                                                          jnp.float32)
    for q in range(9):
        pltpu.sync_copy(rows0, s_acc.at[pl.ds(base + CH * q, CH)])
    pltpu.sync_copy(rows0.at[pl.ds(0, NPT - 9 * CH)],
                    s_acc.at[pl.ds(base + 9 * CH, NPT - 9 * CH)])
    plsc.subcore_barrier()

    gv = gm_l[...]

    def unpack(c, b):
        for g in range(CH // LANES):
            sl = pl.ds(g * LANES, LANES)
            p = pk_l[pl.ds(c * CH + g * LANES, LANES)]
            srcb[b][sl] = p & 0xFFFF
            dstb[b][sl] = lax.shift_right_logical(p, 16)

    def g_desc(b):
        return pltpu.make_async_copy(hp.at[srcb[b]], rowsb[b], gsem)

    def s_desc(b):
        return pltpu.make_async_copy(rowsb[b], s_acc.at[dstb[b]], ssem)

    def compute(c, b):
        rows = rowsb[b]
        for g in range(CH // LANES):
            sl = pl.ds(g * LANES, LANES)
            lane = lax.iota(jnp.int32, LANES)
            asv = plsc.load_gather(
                rows, [lane + g * LANES,
                       jnp.full((LANES,), D + 1, jnp.int32)])
            adv = plsc.load_gather(ad_l, [dstb[b][sl]])
            t = asv + adv
            e = jnp.maximum(t, 0.2 * t)
            z = gv + adv
            mv = jnp.maximum(z, 0.2 * z)
            w = jnp.exp(e - mv)
            eid = c * CH + g * LANES + lane
            w_l[sl] = jnp.where(eid < EPT, w, 0.0)

        @plsc.parallel_loop(0, CH, 1, unroll=4)
        def scale(i):
            wv = plsc.load_gather(w_l, [jnp.full((LANES,), i, jnp.int32)])
            for k in range(DP // LANES):
                ksl = pl.ds(k * LANES, LANES)
                rows[i, ksl] = rows[i, ksl] * wv

    unpack(0, 0)
    g_desc(0).start()

    def pair(i, carry):
        for b in range(2):
            c = 2 * i + b
            g_desc(b).wait()

            @pl.when(c > 0)
            def _():
                s_desc(1 - b).wait()

            @pl.when(c + 1 < NCH)
            def _():
                unpack(c + 1, 1 - b)
                g_desc(1 - b).start()
            compute(c, b)
            s_desc(b).start(add=True)
        return carry
    lax.fori_loop(0, NCH // 2, pair, 0)

    s_desc(1).wait()

    plsc.subcore_barrier()
    pltpu.sync_copy(s_acc.at[pl.ds(base, NPT)],
                    out.at[c_id, pl.ds(base, NPT)])


_edge = pl.kernel(
    _edge_body,
    out_type=jax.ShapeDtypeStruct((NC, N, DP), jnp.float32),
    mesh=plsc.VectorSubcoreMesh(core_axis_name="c", subcore_axis_name="s",
                                num_cores=NC, num_subcores=NS),
    compiler_params=pltpu.CompilerParams(use_tc_tiling_on_sc=False,
                                         needs_layout_passes=False),
    scratch_types=[
        pltpu.VMEM_SHARED((N, DP), jnp.float32),
        pltpu.VMEM((N,), jnp.float32),
        pltpu.VMEM((LANES,), jnp.float32),
        pltpu.VMEM((EPTP,), jnp.int32),
        pltpu.VMEM((CH,), jnp.int32),
        pltpu.VMEM((CH,), jnp.int32),
        pltpu.VMEM((CH,), jnp.int32),
        pltpu.VMEM((CH,), jnp.int32),
        pltpu.VMEM((CH,), jnp.float32),
        pltpu.VMEM((CH, DP), jnp.float32),
        pltpu.VMEM((CH, DP), jnp.float32),
        pltpu.SemaphoreType.DMA,
        pltpu.SemaphoreType.DMA,
    ],
)


@jax.jit
def kernel(x, edge_index, W1, att_src1, att_dst1, b1,
           W2, att_src2, att_dst2, b2):
    packed = jnp.left_shift(edge_index[1], 16) | edge_index[0]
    packed = packed.reshape(NW, EPT)
    packed = jnp.pad(packed, ((0, 0), (0, EPTP - EPT)))

    hp1, ad1, gm1 = _prep1(x, W1, att_src1.reshape(D), att_dst1.reshape(D))
    part1 = _edge(hp1, packed, ad1, gm1)
    hp2, ad2, gm2 = _prep2(part1, b1, W2,
                           att_src2.reshape(D), att_dst2.reshape(D))
    part2 = _edge(hp2, packed, ad2, gm2)
    return _final(part2, b2)
